# Initial kernel scaffold; baseline (speedup 1.0000x reference)
import jax
import jax.numpy as jnp
from jax.experimental import pallas as pl
from jax.experimental.pallas import tpu as pltpu

N = 10000
E = 160000
D_NODE = 256
D_EDGE = 16
H = 4
DH = 128
DE = 32
DOUT = 128
N_CLASSES = 10

BN = 1000
BE = 2000


def _node_proj_body(x_ref, wn_ref, attn_ref, wm_ref, a0_ref, ss_ref, sd_ref):
    x = x_ref[...]
    for h in range(H):
        z = jnp.dot(x, wn_ref[h], preferred_element_type=jnp.float32)
        a0_ref[h] = jnp.dot(z, wm_ref[h, :DH], preferred_element_type=jnp.float32)
        ss_ref[:, h] = z @ attn_ref[h, :DH]
        sd_ref[:, h] = z @ attn_ref[h, DH + DE:]


def _node_proj(xn, Wn0, attn0, Wm0):
    grid = N // BN
    return pl.pallas_call(
        _node_proj_body,
        grid=(grid,),
        in_specs=[
            pl.BlockSpec((BN, D_NODE), lambda i: (i, 0)),
            pl.BlockSpec((H, D_NODE, DH), lambda i: (0, 0, 0)),
            pl.BlockSpec((H, DH + DE + DH), lambda i: (0, 0)),
            pl.BlockSpec((H, DH + DE, DOUT), lambda i: (0, 0, 0)),
        ],
        out_specs=[
            pl.BlockSpec((H, BN, DH), lambda i: (0, i, 0)),
            pl.BlockSpec((BN, H), lambda i: (i, 0)),
            pl.BlockSpec((BN, H), lambda i: (i, 0)),
        ],
        out_shape=[
            jax.ShapeDtypeStruct((H, N, DH), jnp.float32),
            jax.ShapeDtypeStruct((N, H), jnp.float32),
            jax.ShapeDtypeStruct((N, H), jnp.float32),
        ],
    )(xn, Wn0, attn0, Wm0)


def _edge_proj_body(xe_ref, we0_ref, attn0_ref, we1_ref, attn1_ref,
                    ze_ref, se0_ref, se1_ref):
    x = xe_ref[...]
    for h in range(H):
        z = jnp.dot(x, we0_ref[h], preferred_element_type=jnp.float32)
        ze_ref[:, h * DE:(h + 1) * DE] = z
        se0_ref[:, h] = z @ attn0_ref[h, DH:DH + DE]
    ze = ze_ref[...]
    xe1 = jnp.where(ze > 0, ze, 0.01 * ze)
    for h in range(H):
        ve = jnp.dot(we1_ref[h], attn1_ref[h, DH:DH + DE],
                     preferred_element_type=jnp.float32)
        se1_ref[:, h] = xe1 @ ve


def _edge_proj(xe, We0, attn0, We1, attn1):
    grid = E // BE
    return pl.pallas_call(
        _edge_proj_body,
        grid=(grid,),
        in_specs=[
            pl.BlockSpec((BE, D_EDGE), lambda i: (i, 0)),
            pl.BlockSpec((H, D_EDGE, DE), lambda i: (0, 0, 0)),
            pl.BlockSpec((H, DH + DE + DH), lambda i: (0, 0)),
            pl.BlockSpec((H, H * DE, DE), lambda i: (0, 0, 0)),
            pl.BlockSpec((H, DH + DE + DH), lambda i: (0, 0)),
        ],
        out_specs=[
            pl.BlockSpec((BE, H * DE), lambda i: (i, 0)),
            pl.BlockSpec((BE, H), lambda i: (i, 0)),
            pl.BlockSpec((BE, H), lambda i: (i, 0)),
        ],
        out_shape=[
            jax.ShapeDtypeStruct((E, H * DE), jnp.float32),
            jax.ShapeDtypeStruct((E, H), jnp.float32),
            jax.ShapeDtypeStruct((E, H), jnp.float32),
        ],
    )(xe, We0, attn0, We1, attn1)


def _combine_body(accA_ref, accB_ref, den_ref, wm0_ref, wn1_ref, attn1_ref,
                  xn1_ref, ss1_ref, sd1_ref):
    dinv = 1.0 / (den_ref[...] + 1e-9)
    for h in range(H):
        q = accA_ref[:, h * DOUT:(h + 1) * DOUT] * dinv[:, h:h + 1]
        p = accB_ref[:, h * DE:(h + 1) * DE] * dinv[:, h:h + 1]
        nb = q + jnp.dot(p, wm0_ref[h, DH:], preferred_element_type=jnp.float32)
        xn1_ref[:, h * DOUT:(h + 1) * DOUT] = jnp.where(nb > 0, nb, 0.01 * nb)
    x1 = xn1_ref[...]
    for h in range(H):
        vs = jnp.dot(wn1_ref[h], attn1_ref[h, :DH],
                     preferred_element_type=jnp.float32)
        vd = jnp.dot(wn1_ref[h], attn1_ref[h, DH + DE:],
                     preferred_element_type=jnp.float32)
        ss1_ref[:, h] = x1 @ vs
        sd1_ref[:, h] = x1 @ vd


def _combine(accA, accB, denom0, Wm0, Wn1, attn1):
    grid = N // BN
    return pl.pallas_call(
        _combine_body,
        grid=(grid,),
        in_specs=[
            pl.BlockSpec((BN, H * DOUT), lambda i: (i, 0)),
            pl.BlockSpec((BN, H * DE), lambda i: (i, 0)),
            pl.BlockSpec((BN, H), lambda i: (i, 0)),
            pl.BlockSpec((H, DH + DE, DOUT), lambda i: (0, 0, 0)),
            pl.BlockSpec((H, H * DOUT, DH), lambda i: (0, 0, 0)),
            pl.BlockSpec((H, DH + DE + DH), lambda i: (0, 0)),
        ],
        out_specs=[
            pl.BlockSpec((BN, H * DOUT), lambda i: (i, 0)),
            pl.BlockSpec((BN, H), lambda i: (i, 0)),
            pl.BlockSpec((BN, H), lambda i: (i, 0)),
        ],
        out_shape=[
            jax.ShapeDtypeStruct((N, H * DOUT), jnp.float32),
            jax.ShapeDtypeStruct((N, H), jnp.float32),
            jax.ShapeDtypeStruct((N, H), jnp.float32),
        ],
    )(accA, accB, denom0, Wm0, Wn1, attn1)


def _final_body(xn1_ref, c1_ref, ze_ref, alpha1_ref, wn1_ref, we1_ref,
                wm1_ref, fcw_ref, fcb_ref, out_ref, u_acc, w_acc):
    i = pl.program_id(0)
    nsteps = pl.num_programs(0)

    @pl.when(i == 0)
    def _init():
        u_acc[...] = jnp.zeros_like(u_acc)
        w_acc[...] = jnp.zeros_like(w_acc)

    ze = ze_ref[...]
    xe1 = jnp.where(ze > 0, ze, 0.01 * ze)
    w_acc[...] += jnp.dot(alpha1_ref[...].T, xe1,
                          preferred_element_type=jnp.float32)

    @pl.when(i < N // BE)
    def _nodes():
        u_acc[...] += jnp.dot(c1_ref[...].T, xn1_ref[...],
                              preferred_element_type=jnp.float32)

    @pl.when(i == nsteps - 1)
    def _fin():
        u = u_acc[...]
        w = w_acc[...]
        parts = []
        for h in range(H):
            t1 = jnp.dot(jnp.dot(u[h:h + 1], wn1_ref[h],
                                 preferred_element_type=jnp.float32),
                         wm1_ref[h, :DH], preferred_element_type=jnp.float32)
            t2 = jnp.dot(jnp.dot(w[h:h + 1], we1_ref[h],
                                 preferred_element_type=jnp.float32),
                         wm1_ref[h, DH:], preferred_element_type=jnp.float32)
            parts.append(t1 + t2)
        sum_node = jnp.concatenate(parts, axis=1)
        logits = jnp.dot(sum_node, fcw_ref[...],
                         preferred_element_type=jnp.float32) + fcb_ref[...]
        m = jnp.max(logits, axis=1, keepdims=True)
        e = jnp.exp(logits - m)
        out_ref[...] = e / jnp.sum(e, axis=1, keepdims=True)


def _final(xn1, c1, ze0cat, alpha1, Wn1, We1, Wm1, fc_w, fc_b):
    grid = E // BE
    nb = N // BE
    return pl.pallas_call(
        _final_body,
        grid=(grid,),
        in_specs=[
            pl.BlockSpec((BE, H * DOUT), lambda i: (jnp.minimum(i, nb - 1), 0)),
            pl.BlockSpec((BE, H), lambda i: (jnp.minimum(i, nb - 1), 0)),
            pl.BlockSpec((BE, H * DE), lambda i: (i, 0)),
            pl.BlockSpec((BE, H), lambda i: (i, 0)),
            pl.BlockSpec((H, H * DOUT, DH), lambda i: (0, 0, 0)),
            pl.BlockSpec((H, H * DE, DE), lambda i: (0, 0, 0)),
            pl.BlockSpec((H, DH + DE, DOUT), lambda i: (0, 0, 0)),
            pl.BlockSpec((H * DOUT, N_CLASSES), lambda i: (0, 0)),
            pl.BlockSpec((N_CLASSES,), lambda i: (0,)),
        ],
        out_specs=pl.BlockSpec((1, N_CLASSES), lambda i: (0, 0)),
        out_shape=jax.ShapeDtypeStruct((1, N_CLASSES), jnp.float32),
        scratch_shapes=[
            pltpu.VMEM((H, H * DOUT), jnp.float32),
            pltpu.VMEM((H, H * DE), jnp.float32),
        ],
    )(xn1, c1, ze0cat, alpha1, Wn1, We1, Wm1, fc_w, fc_b)


def kernel(xn, xe, edge_index, Wn0, We0, attn0, Wm0, Wn1, We1, attn1, Wm1,
           fc_w, fc_b):
    src = edge_index[0]
    dst = edge_index[1]

    A0, s_src0, s_dst0 = _node_proj(xn, Wn0, attn0, Wm0)
    ze0cat, s_edge0, s_edge1 = _edge_proj(xe, We0, attn0, We1, attn1)

    l0 = s_src0[src] + s_edge0 + s_dst0[dst]
    l0 = jnp.where(l0 > 0, l0, 0.2 * l0)
    ex0 = jnp.exp(l0)
    denom0 = jax.ops.segment_sum(ex0, dst, num_segments=N)
    gA = A0[:, src]
    wA = (ex0.T[:, :, None] * gA).transpose(1, 0, 2).reshape(E, H * DOUT)
    accA = jax.ops.segment_sum(wA, dst, num_segments=N)
    accB = jax.ops.segment_sum(jnp.repeat(ex0, DE, axis=1) * ze0cat, dst,
                               num_segments=N)

    xn1, s_src1, s_dst1 = _combine(accA, accB, denom0, Wm0, Wn1, attn1)

    l1 = s_src1[src] + s_edge1 + s_dst1[dst]
    l1 = jnp.where(l1 > 0, l1, 0.2 * l1)
    ex1 = jnp.exp(l1)
    denom1 = jax.ops.segment_sum(ex1, dst, num_segments=N)
    alpha1 = ex1 / (denom1[dst] + 1e-9)
    c1 = jax.ops.segment_sum(alpha1, src, num_segments=N)

    return _final(xn1, c1, ze0cat, alpha1, Wn1, We1, Wm1, fc_w, fc_b)


# speedup vs baseline: 3.5229x; 3.5229x over previous
import jax
import jax.numpy as jnp
from jax import lax
from jax.experimental import pallas as pl
from jax.experimental.pallas import tpu as pltpu
from jax.experimental.pallas import tpu_sc as plsc

N = 10000
E = 160000
D_NODE = 256
D_EDGE = 16
H = 4
DH = 128
DE = 32
DOUT = 128
N_CLASSES = 10

BN = 1000
BE = 2000

NC = 2
NS = 16
NW = NC * NS
EPT = E // NW
CH = 1000
NCH = EPT // CH
CHP = 1024
NACC = N * H
NACCP = NACC + 64
FC = 128
NFCH = -(-EPT // FC)
NPAD = N + 8
ZR = 16


def _mesh():
    return plsc.VectorSubcoreMesh(core_axis_name="c", subcore_axis_name="s",
                                  num_cores=NC)


def _node_proj_body(x_ref, wn_ref, attn_ref, wm_ref, a0_ref, ss_ref, sd_ref):
    x = x_ref[...]
    for h in range(H):
        z = jnp.dot(x, wn_ref[h], preferred_element_type=jnp.float32)
        a0_ref[h] = jnp.dot(z, wm_ref[h, :DH], preferred_element_type=jnp.float32)
        ss_ref[:, h] = z @ attn_ref[h, :DH]
        sd_ref[:, h] = z @ attn_ref[h, DH + DE:]


def _node_proj(xn, Wn0, attn0, Wm0):
    return pl.pallas_call(
        _node_proj_body,
        grid=(N // BN,),
        in_specs=[
            pl.BlockSpec((BN, D_NODE), lambda i: (i, 0)),
            pl.BlockSpec((H, D_NODE, DH), lambda i: (0, 0, 0)),
            pl.BlockSpec((H, DH + DE + DH), lambda i: (0, 0)),
            pl.BlockSpec((H, DH + DE, DOUT), lambda i: (0, 0, 0)),
        ],
        out_specs=[
            pl.BlockSpec((H, BN, DH), lambda i: (0, i, 0)),
            pl.BlockSpec((BN, H), lambda i: (i, 0)),
            pl.BlockSpec((BN, H), lambda i: (i, 0)),
        ],
        out_shape=[
            jax.ShapeDtypeStruct((H, N, DH), jnp.float32),
            jax.ShapeDtypeStruct((N, H), jnp.float32),
            jax.ShapeDtypeStruct((N, H), jnp.float32),
        ],
    )(xn, Wn0, attn0, Wm0)


def _edge_proj_body(xe_ref, we0_ref, attn0_ref, we1_ref, attn1_ref,
                    ze_ref, se0_ref, se1_ref):
    x = xe_ref[...]
    for h in range(H):
        z = jnp.dot(x, we0_ref[h], preferred_element_type=jnp.float32)
        ze_ref[:, h * DE:(h + 1) * DE] = z
        se0_ref[:, h] = z @ attn0_ref[h, DH:DH + DE]
    ze = ze_ref[...]
    xe1 = jnp.where(ze > 0, ze, 0.01 * ze)
    for h in range(H):
        ve = jnp.dot(we1_ref[h], attn1_ref[h, DH:DH + DE],
                     preferred_element_type=jnp.float32)
        se1_ref[:, h] = xe1 @ ve


def _edge_proj(xe, We0, attn0, We1, attn1):
    return pl.pallas_call(
        _edge_proj_body,
        grid=(E // BE,),
        in_specs=[
            pl.BlockSpec((BE, D_EDGE), lambda i: (i, 0)),
            pl.BlockSpec((H, D_EDGE, DE), lambda i: (0, 0, 0)),
            pl.BlockSpec((H, DH + DE + DH), lambda i: (0, 0)),
            pl.BlockSpec((H, H * DE, DE), lambda i: (0, 0, 0)),
            pl.BlockSpec((H, DH + DE + DH), lambda i: (0, 0)),
        ],
        out_specs=[
            pl.BlockSpec((BE, H * DE), lambda i: (i, 0)),
            pl.BlockSpec((BE, H), lambda i: (i, 0)),
            pl.BlockSpec((BE, H), lambda i: (i, 0)),
        ],
        out_shape=[
            jax.ShapeDtypeStruct((E, H * DE), jnp.float32),
            jax.ShapeDtypeStruct((E, H), jnp.float32),
            jax.ShapeDtypeStruct((E, H), jnp.float32),
        ],
    )(xe, We0, attn0, We1, attn1)


def _sc_scalar_body(tbls_hbm, tbld_hbm, se_hbm, src_hbm, dst_hbm,
                    ex_hbm, den_hbm,
                    tbl_s, tbl_d, srcb, dstb, seb, exb, val128, idx128, acc):
    c = lax.axis_index("c")
    s = lax.axis_index("s")
    wid = s * NC + c
    base = wid * EPT
    it16 = lax.iota(jnp.int32, 16)

    pltpu.sync_copy(tbls_hbm, tbl_s)
    pltpu.sync_copy(tbld_hbm, tbl_d)
    pltpu.sync_copy(src_hbm.at[pl.ds(base, EPT)], srcb.at[pl.ds(0, EPT)])
    pltpu.sync_copy(dst_hbm.at[pl.ds(base, EPT)], dstb.at[pl.ds(0, EPT)])

    def zbody(g, carry):
        exb[pl.ds(g * 16, 16)] = jnp.zeros((16,), jnp.float32)
        return carry
    lax.fori_loop(0, 157, zbody, None)
    pltpu.sync_copy(exb.at[pl.ds(0, 2504)], acc.at[pl.ds(s * 2504, 2504)])
    plsc.subcore_barrier()

    for k in range(NCH):
        koff = k * CH
        pltpu.sync_copy(se_hbm.at[pl.ds((base + koff) * H, CH * H)],
                        seb.at[pl.ds(0, CH * H)])

        def gbody(i, carry):
            for half in range(2):
                slot = i * 32 + half * 16
                lanes = slot + it16
                valid = lanes < CH
                sv = jnp.clip(srcb[pl.ds(koff + slot, 16)], 0, N - 1)
                dv = jnp.clip(dstb[pl.ds(koff + slot, 16)], 0, N - 1)
                dummy = NACC + (lax.rem(i, 4) * 16) + it16
                for h in range(H):
                    pos = lanes * H + h
                    rel = pos - i * 128
                    vs = plsc.load_gather(tbl_s, [sv * H + h])
                    vd = plsc.load_gather(tbl_d, [dv * H + h])
                    se_v = plsc.load_gather(seb, [pos])
                    l = vs + vd + se_v
                    l = jnp.where(l > 0, l, 0.2 * l)
                    ex = jnp.exp(l)
                    plsc.store_scatter(exb, [pos], ex)
                    plsc.store_scatter(val128, [rel], ex)
                    plsc.store_scatter(
                        idx128, [rel], jnp.where(valid, dv * H + h, dummy))
            pltpu.sync_copy(val128, acc.at[idx128], add=True)
            return carry
        lax.fori_loop(0, CHP // 32, gbody, None)

        pltpu.sync_copy(exb.at[pl.ds(0, CH * H)],
                        ex_hbm.at[pl.ds((base + koff) * H, CH * H)])

    plsc.subcore_barrier()
    pltpu.sync_copy(acc.at[pl.ds(s * 2504, 2504)], exb.at[pl.ds(0, 2504)])
    pltpu.sync_copy(exb.at[pl.ds(0, 2504)],
                    den_hbm.at[pl.ds(c * NACCP + s * 2504, 2504)])


def _sc_scalar(tbl_s, tbl_d, s_edge, src, dst):
    kern = pl.kernel(
        _sc_scalar_body,
        out_type=[
            jax.ShapeDtypeStruct((E * H,), jnp.float32),
            jax.ShapeDtypeStruct((NC * NACCP,), jnp.float32),
        ],
        mesh=_mesh(),
        compiler_params=pltpu.CompilerParams(needs_layout_passes=False),
        scratch_types=[
            pltpu.VMEM((NACC,), jnp.float32),
            pltpu.VMEM((NACC,), jnp.float32),
            pltpu.VMEM((EPT + 24,), jnp.int32),
            pltpu.VMEM((EPT + 24,), jnp.int32),
            pltpu.VMEM((CHP * H,), jnp.float32),
            pltpu.VMEM((CHP * H,), jnp.float32),
            pltpu.VMEM((128,), jnp.float32),
            pltpu.VMEM((128,), jnp.int32),
            pltpu.VMEM_SHARED((NACCP,), jnp.float32),
        ],
    )
    return kern(tbl_s, tbl_d, s_edge, src, dst)


def _sc_feature_body(a0_hbm, ze_hbm, ex_hbm, src_hbm, dst_hbm,
                     accA_hbm, accB_hbm,
                     exw, srcb, dstb, rows, zerob, gidx, didx, sem, facc):
    c = lax.axis_index("c")
    s = lax.axis_index("s")
    wid = s * NC + c
    base = wid * EPT
    it16 = lax.iota(jnp.int32, 16)

    pltpu.sync_copy(ex_hbm.at[pl.ds(base * H, EPT * H)],
                    exw.at[pl.ds(0, EPT * H)])
    pltpu.sync_copy(src_hbm.at[pl.ds(base, EPT)], srcb.at[pl.ds(0, EPT)])
    pltpu.sync_copy(dst_hbm.at[pl.ds(base, EPT)], dstb.at[pl.ds(0, EPT)])

    for r in range(ZR):
        for q in range(8):
            zerob[r, pl.ds(q * 16, 16)] = jnp.zeros((16,), jnp.float32)

    rowbase = s * 624
    nzq = jnp.where(s < 15, 39, 40)

    for p in range(H + 1):
        def zcopy(q, carry):
            pltpu.sync_copy(zerob, facc.at[pl.ds(rowbase + q * ZR, ZR)])
            return carry
        lax.fori_loop(0, nzq, zcopy, None)
        plsc.subcore_barrier()

        def chunk(k, carry):
            koff = k * FC

            def idxbody(g, carry2):
                sl = g * 16
                ok = (koff + sl + it16) < EPT
                sv = jnp.clip(srcb[pl.ds(koff + sl, 16)], 0, N - 1)
                dv = jnp.clip(dstb[pl.ds(koff + sl, 16)], 0, N - 1)
                if p < H:
                    gidx[pl.ds(sl, 16)] = jnp.where(ok, sv + p * N, 0)
                else:
                    gidx[pl.ds(sl, 16)] = jnp.clip(
                        base + koff + sl + it16, 0, E - 1)
                didx[pl.ds(sl, 16)] = jnp.where(ok, dv, N + (it16 & 7))
                return carry2
            lax.fori_loop(0, FC // 16, idxbody, None)

            if p < H:
                pltpu.async_copy(a0_hbm.at[gidx], rows, sem).wait()
            else:
                pltpu.async_copy(ze_hbm.at[gidx], rows, sem).wait()

            def scale(i, carry2):
                eoff = (koff + i) * H
                if p < H:
                    scv = [plsc.load_gather(
                        exw, [jnp.full((16,), eoff + p, jnp.int32)])] * H
                else:
                    scv = [plsc.load_gather(
                        exw, [jnp.full((16,), eoff + h, jnp.int32)])
                        for h in range(H)]
                for j in range(8):
                    rows[i, pl.ds(j * 16, 16)] = (
                        rows[i, pl.ds(j * 16, 16)] * scv[j // 2])
                return carry2
            lax.fori_loop(0, FC, scale, None)

            pltpu.sync_copy(rows, facc.at[didx], add=True)
            return carry
        lax.fori_loop(0, NFCH, chunk, None)

        plsc.subcore_barrier()
        out = accA_hbm if p < H else accB_hbm
        obase = ((c * H + p) * N if p < H else c * N)

        def fbody(q, carry):
            pltpu.sync_copy(facc.at[pl.ds(rowbase + q * ZR, ZR)],
                            rows.at[pl.ds(0, ZR)])
            pltpu.sync_copy(rows.at[pl.ds(0, ZR)],
                            out.at[pl.ds(obase + rowbase + q * ZR, ZR)])
            return carry
        lax.fori_loop(0, nzq, fbody, None)
        plsc.subcore_barrier()


def _sc_feature(a0flat, ze0cat, ex0, src, dst):
    kern = pl.kernel(
        _sc_feature_body,
        out_type=[
            jax.ShapeDtypeStruct((NC * H * N, DH), jnp.float32),
            jax.ShapeDtypeStruct((NC * N, DH), jnp.float32),
        ],
        mesh=_mesh(),
        compiler_params=pltpu.CompilerParams(needs_layout_passes=False),
        scratch_types=[
            pltpu.VMEM(((EPT + FC + 16) * H,), jnp.float32),
            pltpu.VMEM((EPT + FC + 16,), jnp.int32),
            pltpu.VMEM((EPT + FC + 16,), jnp.int32),
            pltpu.VMEM((FC, DH), jnp.float32),
            pltpu.VMEM((ZR, DH), jnp.float32),
            pltpu.VMEM((FC,), jnp.int32),
            pltpu.VMEM((FC,), jnp.int32),
            pltpu.SemaphoreType.DMA,
            pltpu.VMEM_SHARED((NPAD, DH), jnp.float32),
        ],
    )
    return kern(a0flat, ze0cat, ex0, src, dst)


def _sc_alpha_body(den_hbm, ex_hbm, src_hbm, dst_hbm,
                   alpha_hbm, c1_hbm,
                   tbl, tbl2, srcb, dstb, exb, val128, idx128, acc):
    c = lax.axis_index("c")
    s = lax.axis_index("s")
    wid = s * NC + c
    base = wid * EPT
    it16 = lax.iota(jnp.int32, 16)

    pltpu.sync_copy(den_hbm.at[pl.ds(0, NACCP)], tbl)
    pltpu.sync_copy(den_hbm.at[pl.ds(NACCP, NACCP)], tbl2)

    def addt(i, carry):
        tbl[pl.ds(i * 16, 16)] = (tbl[pl.ds(i * 16, 16)]
                                  + tbl2[pl.ds(i * 16, 16)])
        return carry
    lax.fori_loop(0, NACCP // 16, addt, None)

    pltpu.sync_copy(src_hbm.at[pl.ds(base, EPT)], srcb.at[pl.ds(0, EPT)])
    pltpu.sync_copy(dst_hbm.at[pl.ds(base, EPT)], dstb.at[pl.ds(0, EPT)])

    def zbody(g, carry):
        exb[pl.ds(g * 16, 16)] = jnp.zeros((16,), jnp.float32)
        return carry
    lax.fori_loop(0, 157, zbody, None)
    pltpu.sync_copy(exb.at[pl.ds(0, 2504)], acc.at[pl.ds(s * 2504, 2504)])
    plsc.subcore_barrier()

    for k in range(NCH):
        koff = k * CH
        pltpu.sync_copy(ex_hbm.at[pl.ds((base + koff) * H, CH * H)],
                        exb.at[pl.ds(0, CH * H)])

        def gbody(i, carry):
            for half in range(2):
                slot = i * 32 + half * 16
                lanes = slot + it16
                valid = lanes < CH
                sv = jnp.clip(srcb[pl.ds(koff + slot, 16)], 0, N - 1)
                dv = jnp.clip(dstb[pl.ds(koff + slot, 16)], 0, N - 1)
                dummy = NACC + (lax.rem(i, 4) * 16) + it16
                for h in range(H):
                    pos = lanes * H + h
                    rel = pos - i * 128
                    den = plsc.load_gather(tbl, [dv * H + h])
                    ex = plsc.load_gather(exb, [pos])
                    al = ex / (den + 1e-9)
                    plsc.store_scatter(exb, [pos], al)
                    plsc.store_scatter(val128, [rel], al)
                    plsc.store_scatter(
                        idx128, [rel], jnp.where(valid, sv * H + h, dummy))
            pltpu.sync_copy(val128, acc.at[idx128], add=True)
            return carry
        lax.fori_loop(0, CHP // 32, gbody, None)

        pltpu.sync_copy(exb.at[pl.ds(0, CH * H)],
                        alpha_hbm.at[pl.ds((base + koff) * H, CH * H)])

    plsc.subcore_barrier()
    pltpu.sync_copy(acc.at[pl.ds(s * 2504, 2504)], exb.at[pl.ds(0, 2504)])
    pltpu.sync_copy(exb.at[pl.ds(0, 2504)],
                    c1_hbm.at[pl.ds(c * NACCP + s * 2504, 2504)])


def _sc_alpha(den_parts, ex1, src, dst):
    kern = pl.kernel(
        _sc_alpha_body,
        out_type=[
            jax.ShapeDtypeStruct((E * H,), jnp.float32),
            jax.ShapeDtypeStruct((NC * NACCP,), jnp.float32),
        ],
        mesh=_mesh(),
        compiler_params=pltpu.CompilerParams(needs_layout_passes=False),
        scratch_types=[
            pltpu.VMEM((NACCP,), jnp.float32),
            pltpu.VMEM((NACCP,), jnp.float32),
            pltpu.VMEM((EPT + 24,), jnp.int32),
            pltpu.VMEM((EPT + 24,), jnp.int32),
            pltpu.VMEM((CHP * H,), jnp.float32),
            pltpu.VMEM((128,), jnp.float32),
            pltpu.VMEM((128,), jnp.int32),
            pltpu.VMEM_SHARED((NACCP,), jnp.float32),
        ],
    )
    return kern(den_parts, ex1, src, dst)


def _combine_body(accA_ref, accB_ref, den_ref, wm0_ref, wn1_ref, attn1_ref,
                  xn1_ref, ss1_ref, sd1_ref):
    den = den_ref[0] + den_ref[1]
    dinv = 1.0 / (den + 1e-9)
    accB = accB_ref[0] + accB_ref[1]
    for h in range(H):
        q = (accA_ref[0, h] + accA_ref[1, h]) * dinv[:, h:h + 1]
        p = accB[:, h * DE:(h + 1) * DE] * dinv[:, h:h + 1]
        nb = q + jnp.dot(p, wm0_ref[h, DH:], preferred_element_type=jnp.float32)
        xn1_ref[:, h * DOUT:(h + 1) * DOUT] = jnp.where(nb > 0, nb, 0.01 * nb)
    x1 = xn1_ref[...]
    for h in range(H):
        vs = jnp.dot(wn1_ref[h], attn1_ref[h, :DH],
                     preferred_element_type=jnp.float32)
        vd = jnp.dot(wn1_ref[h], attn1_ref[h, DH + DE:],
                     preferred_element_type=jnp.float32)
        ss1_ref[:, h] = x1 @ vs
        sd1_ref[:, h] = x1 @ vd


def _combine(accA, accB, den0, Wm0, Wn1, attn1):
    return pl.pallas_call(
        _combine_body,
        grid=(N // BN,),
        in_specs=[
            pl.BlockSpec((NC, H, BN, DH), lambda i: (0, 0, i, 0)),
            pl.BlockSpec((NC, BN, DH), lambda i: (0, i, 0)),
            pl.BlockSpec((NC, BN, H), lambda i: (0, i, 0)),
            pl.BlockSpec((H, DH + DE, DOUT), lambda i: (0, 0, 0)),
            pl.BlockSpec((H, H * DOUT, DH), lambda i: (0, 0, 0)),
            pl.BlockSpec((H, DH + DE + DH), lambda i: (0, 0)),
        ],
        out_specs=[
            pl.BlockSpec((BN, H * DOUT), lambda i: (i, 0)),
            pl.BlockSpec((BN, H), lambda i: (i, 0)),
            pl.BlockSpec((BN, H), lambda i: (i, 0)),
        ],
        out_shape=[
            jax.ShapeDtypeStruct((N, H * DOUT), jnp.float32),
            jax.ShapeDtypeStruct((N, H), jnp.float32),
            jax.ShapeDtypeStruct((N, H), jnp.float32),
        ],
    )(accA, accB, den0, Wm0, Wn1, attn1)


def _final_body(xn1_ref, c1_ref, ze_ref, alpha1_ref, wn1_ref, we1_ref,
                wm1_ref, fcw_ref, fcb_ref, out_ref, u_acc, w_acc):
    i = pl.program_id(0)
    nsteps = pl.num_programs(0)

    @pl.when(i == 0)
    def _init():
        u_acc[...] = jnp.zeros_like(u_acc)
        w_acc[...] = jnp.zeros_like(w_acc)

    ze = ze_ref[...]
    xe1 = jnp.where(ze > 0, ze, 0.01 * ze)
    w_acc[...] += jnp.dot(alpha1_ref[...].T, xe1,
                          preferred_element_type=jnp.float32)

    @pl.when(i < N // BE)
    def _nodes():
        c1 = c1_ref[0] + c1_ref[1]
        u_acc[...] += jnp.dot(c1.T, xn1_ref[...],
                              preferred_element_type=jnp.float32)

    @pl.when(i == nsteps - 1)
    def _fin():
        u = u_acc[...]
        w = w_acc[...]
        parts = []
        for h in range(H):
            t1 = jnp.dot(jnp.dot(u[h:h + 1], wn1_ref[h],
                                 preferred_element_type=jnp.float32),
                         wm1_ref[h, :DH], preferred_element_type=jnp.float32)
            t2 = jnp.dot(jnp.dot(w[h:h + 1], we1_ref[h],
                                 preferred_element_type=jnp.float32),
                         wm1_ref[h, DH:], preferred_element_type=jnp.float32)
            parts.append(t1 + t2)
        sum_node = jnp.concatenate(parts, axis=1)
        logits = jnp.dot(sum_node, fcw_ref[...],
                         preferred_element_type=jnp.float32) + fcb_ref[...]
        m = jnp.max(logits, axis=1, keepdims=True)
        e = jnp.exp(logits - m)
        out_ref[...] = e / jnp.sum(e, axis=1, keepdims=True)


def _final(xn1, c1, ze0cat, alpha1, Wn1, We1, Wm1, fc_w, fc_b):
    nbm = N // BE
    return pl.pallas_call(
        _final_body,
        grid=(E // BE,),
        in_specs=[
            pl.BlockSpec((BE, H * DOUT), lambda i: (jnp.minimum(i, nbm - 1), 0)),
            pl.BlockSpec((NC, BE, H), lambda i: (0, jnp.minimum(i, nbm - 1), 0)),
            pl.BlockSpec((BE, H * DE), lambda i: (i, 0)),
            pl.BlockSpec((BE, H), lambda i: (i, 0)),
            pl.BlockSpec((H, H * DOUT, DH), lambda i: (0, 0, 0)),
            pl.BlockSpec((H, H * DE, DE), lambda i: (0, 0, 0)),
            pl.BlockSpec((H, DH + DE, DOUT), lambda i: (0, 0, 0)),
            pl.BlockSpec((H * DOUT, N_CLASSES), lambda i: (0, 0)),
            pl.BlockSpec((N_CLASSES,), lambda i: (0,)),
        ],
        out_specs=pl.BlockSpec((1, N_CLASSES), lambda i: (0, 0)),
        out_shape=jax.ShapeDtypeStruct((1, N_CLASSES), jnp.float32),
        scratch_shapes=[
            pltpu.VMEM((H, H * DOUT), jnp.float32),
            pltpu.VMEM((H, H * DE), jnp.float32),
        ],
    )(xn1, c1, ze0cat, alpha1, Wn1, We1, Wm1, fc_w, fc_b)


def kernel(xn, xe, edge_index, Wn0, We0, attn0, Wm0, Wn1, We1, attn1, Wm1,
           fc_w, fc_b):
    src = edge_index[0]
    dst = edge_index[1]

    A0, ss0, sd0 = _node_proj(xn, Wn0, attn0, Wm0)
    ze0cat, se0, se1 = _edge_proj(xe, We0, attn0, We1, attn1)

    ex0, den0p = _sc_scalar(ss0.reshape(-1), sd0.reshape(-1),
                            se0.reshape(-1), src, dst)
    accAp, accBp = _sc_feature(A0.reshape(H * N, DH), ze0cat, ex0, src, dst)
    den0 = den0p.reshape(NC, NACCP)[:, :NACC].reshape(NC, N, H)

    xn1, ss1, sd1 = _combine(accAp.reshape(NC, H, N, DH),
                             accBp.reshape(NC, N, DH), den0, Wm0, Wn1, attn1)

    ex1, den1p = _sc_scalar(ss1.reshape(-1), sd1.reshape(-1),
                            se1.reshape(-1), src, dst)
    alpha1f, c1p = _sc_alpha(den1p, ex1, src, dst)
    c1 = c1p.reshape(NC, NACCP)[:, :NACC].reshape(NC, N, H)
    alpha1 = alpha1f.reshape(E, H)

    return _final(xn1, c1, ze0cat, alpha1, Wn1, We1, Wm1, fc_w, fc_b)


# speedup vs baseline: 3.5493x; 1.0075x over previous
import jax
import jax.numpy as jnp
from jax import lax
from jax.experimental import pallas as pl
from jax.experimental.pallas import tpu as pltpu
from jax.experimental.pallas import tpu_sc as plsc

N = 10000
E = 160000
D_NODE = 256
D_EDGE = 16
H = 4
DH = 128
DE = 32
DOUT = 128
N_CLASSES = 10

BN = 1000
BE = 2000

NC = 2
NS = 16
NW = NC * NS
EPT = E // NW
CH = 1000
NCH = EPT // CH
CHP = 1024
NACC = N * H
NACCP = NACC + 64
FC = 128
NFCH = -(-EPT // FC)
NPAD = N + 8
ZR = 16


def _mesh():
    return plsc.VectorSubcoreMesh(core_axis_name="c", subcore_axis_name="s",
                                  num_cores=NC)


def _node_proj_body(x_ref, wn_ref, attn_ref, wm_ref, a0_ref, ss_ref, sd_ref):
    x = x_ref[...]
    for h in range(H):
        z = jnp.dot(x, wn_ref[h], preferred_element_type=jnp.float32)
        a0_ref[h] = jnp.dot(z, wm_ref[h, :DH], preferred_element_type=jnp.float32)
        ss_ref[:, h] = z @ attn_ref[h, :DH]
        sd_ref[:, h] = z @ attn_ref[h, DH + DE:]


def _node_proj(xn, Wn0, attn0, Wm0):
    return pl.pallas_call(
        _node_proj_body,
        grid=(N // BN,),
        in_specs=[
            pl.BlockSpec((BN, D_NODE), lambda i: (i, 0)),
            pl.BlockSpec((H, D_NODE, DH), lambda i: (0, 0, 0)),
            pl.BlockSpec((H, DH + DE + DH), lambda i: (0, 0)),
            pl.BlockSpec((H, DH + DE, DOUT), lambda i: (0, 0, 0)),
        ],
        out_specs=[
            pl.BlockSpec((H, BN, DH), lambda i: (0, i, 0)),
            pl.BlockSpec((BN, H), lambda i: (i, 0)),
            pl.BlockSpec((BN, H), lambda i: (i, 0)),
        ],
        out_shape=[
            jax.ShapeDtypeStruct((H, N, DH), jnp.float32),
            jax.ShapeDtypeStruct((N, H), jnp.float32),
            jax.ShapeDtypeStruct((N, H), jnp.float32),
        ],
    )(xn, Wn0, attn0, Wm0)


def _edge_proj_body(xe_ref, we0_ref, attn0_ref, we1_ref, attn1_ref,
                    ze_ref, se0_ref, se1_ref):
    x = xe_ref[...]
    for h in range(H):
        z = jnp.dot(x, we0_ref[h], preferred_element_type=jnp.float32)
        ze_ref[:, h * DE:(h + 1) * DE] = z
        se0_ref[:, h] = z @ attn0_ref[h, DH:DH + DE]
    ze = ze_ref[...]
    xe1 = jnp.where(ze > 0, ze, 0.01 * ze)
    for h in range(H):
        ve = jnp.dot(we1_ref[h], attn1_ref[h, DH:DH + DE],
                     preferred_element_type=jnp.float32)
        se1_ref[:, h] = xe1 @ ve


def _edge_proj(xe, We0, attn0, We1, attn1):
    return pl.pallas_call(
        _edge_proj_body,
        grid=(E // BE,),
        in_specs=[
            pl.BlockSpec((BE, D_EDGE), lambda i: (i, 0)),
            pl.BlockSpec((H, D_EDGE, DE), lambda i: (0, 0, 0)),
            pl.BlockSpec((H, DH + DE + DH), lambda i: (0, 0)),
            pl.BlockSpec((H, H * DE, DE), lambda i: (0, 0, 0)),
            pl.BlockSpec((H, DH + DE + DH), lambda i: (0, 0)),
        ],
        out_specs=[
            pl.BlockSpec((BE, H * DE), lambda i: (i, 0)),
            pl.BlockSpec((BE, H), lambda i: (i, 0)),
            pl.BlockSpec((BE, H), lambda i: (i, 0)),
        ],
        out_shape=[
            jax.ShapeDtypeStruct((E, H * DE), jnp.float32),
            jax.ShapeDtypeStruct((E, H), jnp.float32),
            jax.ShapeDtypeStruct((E, H), jnp.float32),
        ],
    )(xe, We0, attn0, We1, attn1)


def _sc_scalar_body(tbls_hbm, tbld_hbm, se_hbm, src_hbm, dst_hbm,
                    ex_hbm, den_hbm,
                    tbl_s, tbl_d, srcb, dstb, seb, exb, val128, idx128, acc):
    c = lax.axis_index("c")
    s = lax.axis_index("s")
    wid = s * NC + c
    base = wid * EPT
    it16 = lax.iota(jnp.int32, 16)

    pltpu.sync_copy(tbls_hbm, tbl_s)
    pltpu.sync_copy(tbld_hbm, tbl_d)
    pltpu.sync_copy(src_hbm.at[pl.ds(base, EPT)], srcb.at[pl.ds(0, EPT)])
    pltpu.sync_copy(dst_hbm.at[pl.ds(base, EPT)], dstb.at[pl.ds(0, EPT)])

    def zbody(g, carry):
        exb[pl.ds(g * 16, 16)] = jnp.zeros((16,), jnp.float32)
        return carry
    lax.fori_loop(0, 157, zbody, None)
    pltpu.sync_copy(exb.at[pl.ds(0, 2504)], acc.at[pl.ds(s * 2504, 2504)])
    plsc.subcore_barrier()

    for k in range(NCH):
        koff = k * CH
        pltpu.sync_copy(se_hbm.at[pl.ds((base + koff) * H, CH * H)],
                        seb.at[pl.ds(0, CH * H)])

        def gbody(i, carry):
            for half in range(2):
                slot = i * 32 + half * 16
                lanes = slot + it16
                valid = lanes < CH
                sv = jnp.clip(srcb[pl.ds(koff + slot, 16)], 0, N - 1)
                dv = jnp.clip(dstb[pl.ds(koff + slot, 16)], 0, N - 1)
                dummy = NACC + (lax.rem(i, 4) * 16) + it16
                for h in range(H):
                    pos = lanes * H + h
                    rel = pos - i * 128
                    vs = plsc.load_gather(tbl_s, [sv * H + h])
                    vd = plsc.load_gather(tbl_d, [dv * H + h])
                    se_v = plsc.load_gather(seb, [pos])
                    l = vs + vd + se_v
                    l = jnp.where(l > 0, l, 0.2 * l)
                    ex = jnp.exp(l)
                    plsc.store_scatter(exb, [pos], ex)
                    plsc.store_scatter(val128, [rel], ex)
                    plsc.store_scatter(
                        idx128, [rel], jnp.where(valid, dv * H + h, dummy))
            pltpu.sync_copy(val128, acc.at[idx128], add=True)
            return carry
        lax.fori_loop(0, CHP // 32, gbody, None)

        pltpu.sync_copy(exb.at[pl.ds(0, CH * H)],
                        ex_hbm.at[pl.ds((base + koff) * H, CH * H)])

    plsc.subcore_barrier()
    pltpu.sync_copy(acc.at[pl.ds(s * 2504, 2504)], exb.at[pl.ds(0, 2504)])
    pltpu.sync_copy(exb.at[pl.ds(0, 2504)],
                    den_hbm.at[pl.ds(c * NACCP + s * 2504, 2504)])


def _sc_scalar(tbl_s, tbl_d, s_edge, src, dst):
    kern = pl.kernel(
        _sc_scalar_body,
        out_type=[
            jax.ShapeDtypeStruct((E * H,), jnp.float32),
            jax.ShapeDtypeStruct((NC * NACCP,), jnp.float32),
        ],
        mesh=_mesh(),
        compiler_params=pltpu.CompilerParams(needs_layout_passes=False),
        scratch_types=[
            pltpu.VMEM((NACC,), jnp.float32),
            pltpu.VMEM((NACC,), jnp.float32),
            pltpu.VMEM((EPT + 24,), jnp.int32),
            pltpu.VMEM((EPT + 24,), jnp.int32),
            pltpu.VMEM((CHP * H,), jnp.float32),
            pltpu.VMEM((CHP * H,), jnp.float32),
            pltpu.VMEM((128,), jnp.float32),
            pltpu.VMEM((128,), jnp.int32),
            pltpu.VMEM_SHARED((NACCP,), jnp.float32),
        ],
    )
    return kern(tbl_s, tbl_d, s_edge, src, dst)


def _sc_feature_body(a0_hbm, ze_hbm, ex_hbm, src_hbm, dst_hbm,
                     accA_hbm, accB_hbm,
                     exw, srcb, dstb, rows, zerob, gidx, didx, sem, facc):
    c = lax.axis_index("c")
    s = lax.axis_index("s")
    wid = s * NC + c
    base = wid * EPT
    it16 = lax.iota(jnp.int32, 16)

    pltpu.sync_copy(ex_hbm.at[pl.ds(base * H, EPT * H)],
                    exw.at[pl.ds(0, EPT * H)])
    pltpu.sync_copy(src_hbm.at[pl.ds(base, EPT)], srcb.at[pl.ds(0, EPT)])
    pltpu.sync_copy(dst_hbm.at[pl.ds(base, EPT)], dstb.at[pl.ds(0, EPT)])

    for r in range(ZR):
        for q in range(8):
            zerob[r, pl.ds(q * 16, 16)] = jnp.zeros((16,), jnp.float32)

    rowbase = s * 624
    nzq = jnp.where(s < 15, 39, 40)

    for p in range(H + 1):
        def zcopy(q, carry):
            pltpu.sync_copy(zerob, facc.at[pl.ds(rowbase + q * ZR, ZR)])
            return carry
        lax.fori_loop(0, nzq, zcopy, None)
        plsc.subcore_barrier()

        def chunk(k, carry):
            koff = k * FC

            def idxbody(g, carry2):
                sl = g * 16
                ok = (koff + sl + it16) < EPT
                sv = jnp.clip(srcb[pl.ds(koff + sl, 16)], 0, N - 1)
                dv = jnp.clip(dstb[pl.ds(koff + sl, 16)], 0, N - 1)
                if p < H:
                    gidx[pl.ds(sl, 16)] = jnp.where(ok, sv + p * N, 0)
                else:
                    gidx[pl.ds(sl, 16)] = jnp.clip(
                        base + koff + sl + it16, 0, E - 1)
                didx[pl.ds(sl, 16)] = jnp.where(ok, dv, N + (it16 & 7))
                return carry2
            lax.fori_loop(0, FC // 16, idxbody, None)

            if p < H:
                pltpu.async_copy(a0_hbm.at[gidx], rows, sem).wait()
            else:
                pltpu.async_copy(ze_hbm.at[gidx], rows, sem).wait()

            def scale(i2, carry2):
                for u in range(2):
                    i = i2 * 2 + u
                    eoff = (koff + i) * H
                    if p < H:
                        scv = [plsc.load_gather(
                            exw, [jnp.full((16,), eoff + p, jnp.int32)])] * H
                    else:
                        scv = [plsc.load_gather(
                            exw, [jnp.full((16,), eoff + h, jnp.int32)])
                            for h in range(H)]
                    for j in range(8):
                        rows[i, pl.ds(j * 16, 16)] = (
                            rows[i, pl.ds(j * 16, 16)] * scv[j // 2])
                return carry2
            lax.fori_loop(0, FC // 2, scale, None)

            pltpu.sync_copy(rows, facc.at[didx], add=True)
            return carry
        lax.fori_loop(0, NFCH, chunk, None)

        plsc.subcore_barrier()
        out = accA_hbm if p < H else accB_hbm
        obase = ((c * H + p) * N if p < H else c * N)

        def fbody(q, carry):
            pltpu.sync_copy(facc.at[pl.ds(rowbase + q * ZR, ZR)],
                            rows.at[pl.ds(0, ZR)])
            pltpu.sync_copy(rows.at[pl.ds(0, ZR)],
                            out.at[pl.ds(obase + rowbase + q * ZR, ZR)])
            return carry
        lax.fori_loop(0, nzq, fbody, None)
        plsc.subcore_barrier()


def _sc_feature(a0flat, ze0cat, ex0, src, dst):
    kern = pl.kernel(
        _sc_feature_body,
        out_type=[
            jax.ShapeDtypeStruct((NC * H * N, DH), jnp.float32),
            jax.ShapeDtypeStruct((NC * N, DH), jnp.float32),
        ],
        mesh=_mesh(),
        compiler_params=pltpu.CompilerParams(needs_layout_passes=False),
        scratch_types=[
            pltpu.VMEM(((EPT + FC + 16) * H,), jnp.float32),
            pltpu.VMEM((EPT + FC + 16,), jnp.int32),
            pltpu.VMEM((EPT + FC + 16,), jnp.int32),
            pltpu.VMEM((FC, DH), jnp.float32),
            pltpu.VMEM((ZR, DH), jnp.float32),
            pltpu.VMEM((FC,), jnp.int32),
            pltpu.VMEM((FC,), jnp.int32),
            pltpu.SemaphoreType.DMA,
            pltpu.VMEM_SHARED((NPAD, DH), jnp.float32),
        ],
    )
    return kern(a0flat, ze0cat, ex0, src, dst)


def _sc_alpha_body(den_hbm, ex_hbm, src_hbm, dst_hbm,
                   alpha_hbm, c1_hbm,
                   tbl, tbl2, srcb, dstb, exb, val128, idx128, acc):
    c = lax.axis_index("c")
    s = lax.axis_index("s")
    wid = s * NC + c
    base = wid * EPT
    it16 = lax.iota(jnp.int32, 16)

    pltpu.sync_copy(den_hbm.at[pl.ds(0, NACCP)], tbl)
    pltpu.sync_copy(den_hbm.at[pl.ds(NACCP, NACCP)], tbl2)

    def addt(i, carry):
        tbl[pl.ds(i * 16, 16)] = (tbl[pl.ds(i * 16, 16)]
                                  + tbl2[pl.ds(i * 16, 16)])
        return carry
    lax.fori_loop(0, NACCP // 16, addt, None)

    pltpu.sync_copy(src_hbm.at[pl.ds(base, EPT)], srcb.at[pl.ds(0, EPT)])
    pltpu.sync_copy(dst_hbm.at[pl.ds(base, EPT)], dstb.at[pl.ds(0, EPT)])

    def zbody(g, carry):
        exb[pl.ds(g * 16, 16)] = jnp.zeros((16,), jnp.float32)
        return carry
    lax.fori_loop(0, 157, zbody, None)
    pltpu.sync_copy(exb.at[pl.ds(0, 2504)], acc.at[pl.ds(s * 2504, 2504)])
    plsc.subcore_barrier()

    for k in range(NCH):
        koff = k * CH
        pltpu.sync_copy(ex_hbm.at[pl.ds((base + koff) * H, CH * H)],
                        exb.at[pl.ds(0, CH * H)])

        def gbody(i, carry):
            for half in range(2):
                slot = i * 32 + half * 16
                lanes = slot + it16
                valid = lanes < CH
                sv = jnp.clip(srcb[pl.ds(koff + slot, 16)], 0, N - 1)
                dv = jnp.clip(dstb[pl.ds(koff + slot, 16)], 0, N - 1)
                dummy = NACC + (lax.rem(i, 4) * 16) + it16
                for h in range(H):
                    pos = lanes * H + h
                    rel = pos - i * 128
                    den = plsc.load_gather(tbl, [dv * H + h])
                    ex = plsc.load_gather(exb, [pos])
                    al = ex / (den + 1e-9)
                    plsc.store_scatter(exb, [pos], al)
                    plsc.store_scatter(val128, [rel], al)
                    plsc.store_scatter(
                        idx128, [rel], jnp.where(valid, sv * H + h, dummy))
            pltpu.sync_copy(val128, acc.at[idx128], add=True)
            return carry
        lax.fori_loop(0, CHP // 32, gbody, None)

        pltpu.sync_copy(exb.at[pl.ds(0, CH * H)],
                        alpha_hbm.at[pl.ds((base + koff) * H, CH * H)])

    plsc.subcore_barrier()
    pltpu.sync_copy(acc.at[pl.ds(s * 2504, 2504)], exb.at[pl.ds(0, 2504)])
    pltpu.sync_copy(exb.at[pl.ds(0, 2504)],
                    c1_hbm.at[pl.ds(c * NACCP + s * 2504, 2504)])


def _sc_alpha(den_parts, ex1, src, dst):
    kern = pl.kernel(
        _sc_alpha_body,
        out_type=[
            jax.ShapeDtypeStruct((E * H,), jnp.float32),
            jax.ShapeDtypeStruct((NC * NACCP,), jnp.float32),
        ],
        mesh=_mesh(),
        compiler_params=pltpu.CompilerParams(needs_layout_passes=False),
        scratch_types=[
            pltpu.VMEM((NACCP,), jnp.float32),
            pltpu.VMEM((NACCP,), jnp.float32),
            pltpu.VMEM((EPT + 24,), jnp.int32),
            pltpu.VMEM((EPT + 24,), jnp.int32),
            pltpu.VMEM((CHP * H,), jnp.float32),
            pltpu.VMEM((128,), jnp.float32),
            pltpu.VMEM((128,), jnp.int32),
            pltpu.VMEM_SHARED((NACCP,), jnp.float32),
        ],
    )
    return kern(den_parts, ex1, src, dst)


def _combine_body(accA_ref, accB_ref, den_ref, wm0_ref, wn1_ref, attn1_ref,
                  xn1_ref, ss1_ref, sd1_ref):
    den = den_ref[0] + den_ref[1]
    dinv = 1.0 / (den + 1e-9)
    accB = accB_ref[0] + accB_ref[1]
    for h in range(H):
        q = (accA_ref[0, h] + accA_ref[1, h]) * dinv[:, h:h + 1]
        p = accB[:, h * DE:(h + 1) * DE] * dinv[:, h:h + 1]
        nb = q + jnp.dot(p, wm0_ref[h, DH:], preferred_element_type=jnp.float32)
        xn1_ref[:, h * DOUT:(h + 1) * DOUT] = jnp.where(nb > 0, nb, 0.01 * nb)
    x1 = xn1_ref[...]
    for h in range(H):
        vs = jnp.dot(wn1_ref[h], attn1_ref[h, :DH],
                     preferred_element_type=jnp.float32)
        vd = jnp.dot(wn1_ref[h], attn1_ref[h, DH + DE:],
                     preferred_element_type=jnp.float32)
        ss1_ref[:, h] = x1 @ vs
        sd1_ref[:, h] = x1 @ vd


def _combine(accA, accB, den0, Wm0, Wn1, attn1):
    return pl.pallas_call(
        _combine_body,
        grid=(N // BN,),
        in_specs=[
            pl.BlockSpec((NC, H, BN, DH), lambda i: (0, 0, i, 0)),
            pl.BlockSpec((NC, BN, DH), lambda i: (0, i, 0)),
            pl.BlockSpec((NC, BN, H), lambda i: (0, i, 0)),
            pl.BlockSpec((H, DH + DE, DOUT), lambda i: (0, 0, 0)),
            pl.BlockSpec((H, H * DOUT, DH), lambda i: (0, 0, 0)),
            pl.BlockSpec((H, DH + DE + DH), lambda i: (0, 0)),
        ],
        out_specs=[
            pl.BlockSpec((BN, H * DOUT), lambda i: (i, 0)),
            pl.BlockSpec((BN, H), lambda i: (i, 0)),
            pl.BlockSpec((BN, H), lambda i: (i, 0)),
        ],
        out_shape=[
            jax.ShapeDtypeStruct((N, H * DOUT), jnp.float32),
            jax.ShapeDtypeStruct((N, H), jnp.float32),
            jax.ShapeDtypeStruct((N, H), jnp.float32),
        ],
    )(accA, accB, den0, Wm0, Wn1, attn1)


def _final_body(xn1_ref, c1_ref, ze_ref, alpha1_ref, wn1_ref, we1_ref,
                wm1_ref, fcw_ref, fcb_ref, out_ref, u_acc, w_acc):
    i = pl.program_id(0)
    nsteps = pl.num_programs(0)

    @pl.when(i == 0)
    def _init():
        u_acc[...] = jnp.zeros_like(u_acc)
        w_acc[...] = jnp.zeros_like(w_acc)

    ze = ze_ref[...]
    xe1 = jnp.where(ze > 0, ze, 0.01 * ze)
    w_acc[...] += jnp.dot(alpha1_ref[...].T, xe1,
                          preferred_element_type=jnp.float32)

    @pl.when(i < N // BE)
    def _nodes():
        c1 = c1_ref[0] + c1_ref[1]
        u_acc[...] += jnp.dot(c1.T, xn1_ref[...],
                              preferred_element_type=jnp.float32)

    @pl.when(i == nsteps - 1)
    def _fin():
        u = u_acc[...]
        w = w_acc[...]
        parts = []
        for h in range(H):
            t1 = jnp.dot(jnp.dot(u[h:h + 1], wn1_ref[h],
                                 preferred_element_type=jnp.float32),
                         wm1_ref[h, :DH], preferred_element_type=jnp.float32)
            t2 = jnp.dot(jnp.dot(w[h:h + 1], we1_ref[h],
                                 preferred_element_type=jnp.float32),
                         wm1_ref[h, DH:], preferred_element_type=jnp.float32)
            parts.append(t1 + t2)
        sum_node = jnp.concatenate(parts, axis=1)
        logits = jnp.dot(sum_node, fcw_ref[...],
                         preferred_element_type=jnp.float32) + fcb_ref[...]
        m = jnp.max(logits, axis=1, keepdims=True)
        e = jnp.exp(logits - m)
        out_ref[...] = e / jnp.sum(e, axis=1, keepdims=True)


def _final(xn1, c1, ze0cat, alpha1, Wn1, We1, Wm1, fc_w, fc_b):
    nbm = N // BE
    return pl.pallas_call(
        _final_body,
        grid=(E // BE,),
        in_specs=[
            pl.BlockSpec((BE, H * DOUT), lambda i: (jnp.minimum(i, nbm - 1), 0)),
            pl.BlockSpec((NC, BE, H), lambda i: (0, jnp.minimum(i, nbm - 1), 0)),
            pl.BlockSpec((BE, H * DE), lambda i: (i, 0)),
            pl.BlockSpec((BE, H), lambda i: (i, 0)),
            pl.BlockSpec((H, H * DOUT, DH), lambda i: (0, 0, 0)),
            pl.BlockSpec((H, H * DE, DE), lambda i: (0, 0, 0)),
            pl.BlockSpec((H, DH + DE, DOUT), lambda i: (0, 0, 0)),
            pl.BlockSpec((H * DOUT, N_CLASSES), lambda i: (0, 0)),
            pl.BlockSpec((N_CLASSES,), lambda i: (0,)),
        ],
        out_specs=pl.BlockSpec((1, N_CLASSES), lambda i: (0, 0)),
        out_shape=jax.ShapeDtypeStruct((1, N_CLASSES), jnp.float32),
        scratch_shapes=[
            pltpu.VMEM((H, H * DOUT), jnp.float32),
            pltpu.VMEM((H, H * DE), jnp.float32),
        ],
    )(xn1, c1, ze0cat, alpha1, Wn1, We1, Wm1, fc_w, fc_b)


def kernel(xn, xe, edge_index, Wn0, We0, attn0, Wm0, Wn1, We1, attn1, Wm1,
           fc_w, fc_b):
    src = edge_index[0]
    dst = edge_index[1]

    A0, ss0, sd0 = _node_proj(xn, Wn0, attn0, Wm0)
    ze0cat, se0, se1 = _edge_proj(xe, We0, attn0, We1, attn1)

    ex0, den0p = _sc_scalar(ss0.reshape(-1), sd0.reshape(-1),
                            se0.reshape(-1), src, dst)
    accAp, accBp = _sc_feature(A0.reshape(H * N, DH), ze0cat, ex0, src, dst)
    den0 = den0p.reshape(NC, NACCP)[:, :NACC].reshape(NC, N, H)

    xn1, ss1, sd1 = _combine(accAp.reshape(NC, H, N, DH),
                             accBp.reshape(NC, N, DH), den0, Wm0, Wn1, attn1)

    ex1, den1p = _sc_scalar(ss1.reshape(-1), sd1.reshape(-1),
                            se1.reshape(-1), src, dst)
    alpha1f, c1p = _sc_alpha(den1p, ex1, src, dst)
    c1 = c1p.reshape(NC, NACCP)[:, :NACC].reshape(NC, N, H)
    alpha1 = alpha1f.reshape(E, H)

    return _final(xn1, c1, ze0cat, alpha1, Wn1, We1, Wm1, fc_w, fc_b)
